# baseline (speedup 1.0000x reference)
import functools

import jax
import jax.numpy as jnp
import numpy as np
from jax import lax
from jax.experimental import pallas as pl
from jax.experimental.pallas import tpu as pltpu
from jax.experimental.pallas import tpu_sc as plsc

B = 1024
MEM = 50
SENT = 20
DIM = 32
VOCAB = 100000
HOPS = 3

NC = 2
NS = 16
NW = NC * NS

NSPLIT = 2
BH = B // NSPLIT
BM = BH * MEM
PAIRS_W = BM // NW
CHUNK_PAIRS = 32
ROWS_CHUNK = CHUNK_PAIRS * SENT
NSTREAM = ROWS_CHUNK // 128
NCHUNK = PAIRS_W // CHUNK_PAIRS

HALF = DIM // 2

QPAIRS = BH // NW
QROWS = QPAIRS * SENT


def _pos_encoding():
    i = np.arange(1, DIM + 1, dtype=np.float32)[:, None]
    j = np.arange(1, SENT + 1, dtype=np.float32)[None, :]
    e = (i - (DIM + 1) / 2.0) * (j - (SENT + 1) / 2.0)
    e = 1 + 4 * e / DIM / SENT
    e[:, -1] = 1.0
    return np.transpose(e).astype(np.float32)


_ENC = _pos_encoding()

_PERM = np.concatenate([np.arange(0, DIM, 2), np.arange(1, DIM, 2)])
_INV_PERM = np.argsort(_PERM)
_ENC_PERM = _ENC[:, _PERM].copy()


def _sc_body(story_h, query_h, c0, c1, c2, c3, enc_h,
             m0_out, m1_out, m2_out, m3_out, u0_out,
             idx_v, rows_v, enc_v, out_v, sem):
    cid = lax.axis_index("c")
    sid = lax.axis_index("s")
    wid = cid * NS + sid

    pltpu.sync_copy(enc_h, enc_v)
    enc_vals = [(enc_v[s, pl.ds(0, HALF)], enc_v[s, pl.ds(HALF, HALF)])
                for s in range(SENT)]
    tables = (c0, c1, c2, c3)
    m_outs = (m0_out, m1_out, m2_out, m3_out)

    def weighted_sums(t, npairs):
        mask_hi = jnp.full((HALF,), -65536, dtype=jnp.int32)

        def body(p, carry):
            base = p * SENT
            acc0 = jnp.zeros((HALF,), jnp.float32)
            acc1 = jnp.zeros((HALF,), jnp.float32)
            for s in range(SENT):
                rbits = rows_v[t, base + s, :]
                lo = lax.bitcast_convert_type(
                    lax.shift_left(rbits, 16), jnp.float32)
                hi = lax.bitcast_convert_type(
                    lax.bitwise_and(rbits, mask_hi), jnp.float32)
                acc0 = acc0 + lo * enc_vals[s][0]
                acc1 = acc1 + hi * enc_vals[s][1]
            out_v[p, pl.ds(0, HALF)] = acc0
            out_v[p, pl.ds(HALF, HALF)] = acc1
            return carry
        lax.fori_loop(0, npairs, body, 0)

    def chunk_body(c, carry):
        pltpu.sync_copy(
            story_h.at[pl.ds(wid * PAIRS_W * SENT + c * ROWS_CHUNK, ROWS_CHUNK)],
            idx_v)
        cps = []
        for t in range(4):
            for k in range(NSTREAM):
                cps.append(pltpu.async_copy(
                    tables[t].at[idx_v.at[pl.ds(k * 128, 128)]],
                    rows_v.at[t, pl.ds(k * 128, 128)], sem))
        for cp in cps:
            cp.wait()
        for t in range(4):
            weighted_sums(t, CHUNK_PAIRS)
            pltpu.sync_copy(
                out_v,
                m_outs[t].at[pl.ds(wid * PAIRS_W + c * CHUNK_PAIRS, CHUNK_PAIRS)])
        return carry

    lax.fori_loop(0, NCHUNK, chunk_body, 0)

    pltpu.sync_copy(query_h.at[pl.ds(wid * QROWS, QROWS)],
                    idx_v.at[pl.ds(0, QROWS)])
    qcps = [pltpu.async_copy(c0.at[idx_v.at[pl.ds(k * 64, 64)]],
                             rows_v.at[0, pl.ds(k * 64, 64)], sem)
            for k in range(QROWS // 64)]
    for cp in qcps:
        cp.wait()
    weighted_sums(0, QPAIRS)
    pltpu.sync_copy(out_v.at[pl.ds(0, QPAIRS)],
                    u0_out.at[pl.ds(wid * QPAIRS, QPAIRS)])


_sc_gather = functools.partial(
    pl.kernel,
    out_type=[
        jax.ShapeDtypeStruct((BM, DIM), jnp.float32),
        jax.ShapeDtypeStruct((BM, DIM), jnp.float32),
        jax.ShapeDtypeStruct((BM, DIM), jnp.float32),
        jax.ShapeDtypeStruct((BM, DIM), jnp.float32),
        jax.ShapeDtypeStruct((BH, DIM), jnp.float32),
    ],
    mesh=plsc.VectorSubcoreMesh(core_axis_name="c", subcore_axis_name="s"),
    compiler_params=pltpu.CompilerParams(use_tc_tiling_on_sc=False),
    scratch_types=[
        pltpu.VMEM((ROWS_CHUNK,), jnp.int32),
        pltpu.VMEM((4, ROWS_CHUNK, HALF), jnp.int32),
        pltpu.VMEM((SENT, DIM), jnp.float32),
        pltpu.VMEM((CHUNK_PAIRS, DIM), jnp.float32),
        pltpu.SemaphoreType.DMA,
    ],
)(_sc_body)


BBLK = 128


def _hops_body(u0_ref, m0_ref, m1_ref, m2_ref, m3_ref, twt_ref, tb_ref, u_ref):
    u = u0_ref[...]
    twt = twt_ref[...]
    tb = tb_ref[...]
    ms = (m0_ref, m1_ref, m2_ref, m3_ref)
    for h in range(HOPS):
        mh = ms[h][...]
        scores = jnp.sum(mh * u[:, None, :], axis=2)
        smax = jnp.max(scores, axis=1, keepdims=True)
        e = jnp.exp(scores - smax)
        p = e / jnp.sum(e, axis=1, keepdims=True)
        mc = ms[h + 1][...]
        o = jnp.sum(mc * p[:, :, None], axis=1)
        z = jnp.dot(u, twt, preferred_element_type=jnp.float32) + tb
        t = 1.0 / (1.0 + jnp.exp(-z))
        u = (1.0 - t) * u + o * t
    u_ref[...] = u


VBLK = 2048
NV = (VOCAB + VBLK - 1) // VBLK


def _mm_body_a(u_ref, c3_ref, o_ref):
    o_ref[...] = lax.dot_general(
        c3_ref[...], u_ref[...], (((1,), (1,)), ((), ())),
        preferred_element_type=jnp.float32)


def _mm_body_b(u_ref, c3_ref, _ahat_ref, o_ref):
    o_ref[...] = lax.dot_general(
        c3_ref[...], u_ref[...], (((1,), (1,)), ((), ())),
        preferred_element_type=jnp.float32)


def _hops(u0, m0, m1, m2, m3, twt, tb2):
    mspec = pl.BlockSpec((BBLK, MEM, DIM), lambda b: (b, 0, 0))
    return pl.pallas_call(
        _hops_body,
        grid=(BH // BBLK,),
        in_specs=[
            pl.BlockSpec((BBLK, DIM), lambda b: (b, 0)),
            mspec, mspec, mspec, mspec,
            pl.BlockSpec((DIM, DIM), lambda b: (0, 0)),
            pl.BlockSpec((1, DIM), lambda b: (0, 0)),
        ],
        out_specs=pl.BlockSpec((BBLK, DIM), lambda b: (b, 0)),
        out_shape=jax.ShapeDtypeStruct((BH, DIM), jnp.float32),
    )(u0, m0.reshape(BH, MEM, DIM), m1.reshape(BH, MEM, DIM),
      m2.reshape(BH, MEM, DIM), m3.reshape(BH, MEM, DIM), twt, tb2)


def kernel(story, query, C0, C1, C2, C3, Tk_w, Tk_b):
    enc = jnp.asarray(_ENC_PERM)
    perm = jnp.asarray(_PERM)
    inv = jnp.asarray(_INV_PERM)
    twt = Tk_w.T[perm][:, perm]
    tb2 = Tk_b[perm].reshape(1, DIM)
    cb = [lax.bitcast_convert_type(
              c.astype(jnp.bfloat16).reshape(VOCAB, HALF, 2), jnp.int32)
          for c in (C0, C1, C2, C3)]

    scs = []
    for i in range(NSPLIT):
        story_i = story[i * BH:(i + 1) * BH].reshape(-1).astype(jnp.int32)
        query_i = query[i * BH:(i + 1) * BH].reshape(-1).astype(jnp.int32)
        scs.append(_sc_gather(story_i, query_i, cb[0], cb[1], cb[2], cb[3], enc))

    us = [_hops(sc[4], sc[0], sc[1], sc[2], sc[3], twt, tb2)[:, inv]
          for sc in scs]

    a_hat_t = pl.pallas_call(
        _mm_body_a,
        grid=(NV,),
        in_specs=[
            pl.BlockSpec((BH, DIM), lambda j: (0, 0)),
            pl.BlockSpec((VBLK, DIM), lambda j: (j, 0)),
        ],
        out_specs=pl.BlockSpec((VBLK, BH), lambda j: (j, 0)),
        out_shape=jax.ShapeDtypeStruct((VOCAB, B), jnp.float32),
    )(us[0], C3)
    for i in range(1, NSPLIT):
        a_hat_t = pl.pallas_call(
            _mm_body_b,
            grid=(NV,),
            in_specs=[
                pl.BlockSpec((BH, DIM), lambda j: (0, 0)),
                pl.BlockSpec((VBLK, DIM), lambda j: (j, 0)),
                pl.BlockSpec(memory_space=pl.ANY),
            ],
            out_specs=pl.BlockSpec((VBLK, BH), lambda j, i=i: (j, i)),
            out_shape=jax.ShapeDtypeStruct((VOCAB, B), jnp.float32),
            input_output_aliases={2: 0},
        )(us[i], C3, a_hat_t)
    return a_hat_t.T


# speedup vs baseline: 1.4852x; 1.4852x over previous
import functools

import jax
import jax.numpy as jnp
import numpy as np
from jax import lax
from jax.experimental import pallas as pl
from jax.experimental.pallas import tpu as pltpu
from jax.experimental.pallas import tpu_sc as plsc

B = 1024
MEM = 50
SENT = 20
DIM = 32
VOCAB = 100000
HOPS = 3

NC = 2
NS = 16
NW = NC * NS

NSPLIT = 2
BH = B // NSPLIT
BM = BH * MEM
PAIRS_W = BM // NW
CHUNK_PAIRS = 32
ROWS_CHUNK = CHUNK_PAIRS * SENT
NSTREAM = ROWS_CHUNK // 128
NCHUNK = PAIRS_W // CHUNK_PAIRS

HALF = DIM // 2

QPAIRS = BH // NW
QROWS = QPAIRS * SENT


def _pos_encoding():
    i = np.arange(1, DIM + 1, dtype=np.float32)[:, None]
    j = np.arange(1, SENT + 1, dtype=np.float32)[None, :]
    e = (i - (DIM + 1) / 2.0) * (j - (SENT + 1) / 2.0)
    e = 1 + 4 * e / DIM / SENT
    e[:, -1] = 1.0
    return np.transpose(e).astype(np.float32)


_ENC = _pos_encoding()


def _sc_body(story_h, query_h, c0, c1, c2, c3, enc_h,
             m0_out, m1_out, m2_out, m3_out, u0_out,
             idx_v, rows_v, enc_v, out_v, sem):
    cid = lax.axis_index("c")
    sid = lax.axis_index("s")
    wid = cid * NS + sid

    pltpu.sync_copy(enc_h, enc_v)
    enc_vals = [(enc_v[s, pl.ds(0, HALF)], enc_v[s, pl.ds(HALF, HALF)])
                for s in range(SENT)]
    tables = (c0, c1, c2, c3)
    m_outs = (m0_out, m1_out, m2_out, m3_out)

    def weighted_sums(t, npairs):
        def body(p, carry):
            base = p * SENT
            acc0 = rows_v[t, base, pl.ds(0, HALF)] * enc_vals[0][0]
            acc1 = rows_v[t, base, pl.ds(HALF, HALF)] * enc_vals[0][1]
            for s in range(1, SENT):
                acc0 = acc0 + rows_v[t, base + s, pl.ds(0, HALF)] * enc_vals[s][0]
                acc1 = acc1 + rows_v[t, base + s, pl.ds(HALF, HALF)] * enc_vals[s][1]
            out_v[p, pl.ds(0, HALF)] = acc0
            out_v[p, pl.ds(HALF, HALF)] = acc1
            return carry
        lax.fori_loop(0, npairs, body, 0)

    def chunk_body(c, carry):
        pltpu.sync_copy(
            story_h.at[pl.ds(wid * PAIRS_W * SENT + c * ROWS_CHUNK, ROWS_CHUNK)],
            idx_v)
        cps = []
        for t in range(4):
            for k in range(NSTREAM):
                cps.append(pltpu.async_copy(
                    tables[t].at[idx_v.at[pl.ds(k * 128, 128)]],
                    rows_v.at[t, pl.ds(k * 128, 128)], sem))
        for cp in cps:
            cp.wait()
        for t in range(4):
            weighted_sums(t, CHUNK_PAIRS)
            pltpu.sync_copy(
                out_v,
                m_outs[t].at[pl.ds(wid * PAIRS_W + c * CHUNK_PAIRS, CHUNK_PAIRS)])
        return carry

    lax.fori_loop(0, NCHUNK, chunk_body, 0)

    pltpu.sync_copy(query_h.at[pl.ds(wid * QROWS, QROWS)],
                    idx_v.at[pl.ds(0, QROWS)])
    qcps = [pltpu.async_copy(c0.at[idx_v.at[pl.ds(k * 64, 64)]],
                             rows_v.at[0, pl.ds(k * 64, 64)], sem)
            for k in range(QROWS // 64)]
    for cp in qcps:
        cp.wait()
    weighted_sums(0, QPAIRS)
    pltpu.sync_copy(out_v.at[pl.ds(0, QPAIRS)],
                    u0_out.at[pl.ds(wid * QPAIRS, QPAIRS)])


_sc_gather = functools.partial(
    pl.kernel,
    out_type=[
        jax.ShapeDtypeStruct((BM, DIM), jnp.float32),
        jax.ShapeDtypeStruct((BM, DIM), jnp.float32),
        jax.ShapeDtypeStruct((BM, DIM), jnp.float32),
        jax.ShapeDtypeStruct((BM, DIM), jnp.float32),
        jax.ShapeDtypeStruct((BH, DIM), jnp.float32),
    ],
    mesh=plsc.VectorSubcoreMesh(core_axis_name="c", subcore_axis_name="s"),
    compiler_params=pltpu.CompilerParams(use_tc_tiling_on_sc=False),
    scratch_types=[
        pltpu.VMEM((ROWS_CHUNK,), jnp.int32),
        pltpu.VMEM((4, ROWS_CHUNK, DIM), jnp.float32),
        pltpu.VMEM((SENT, DIM), jnp.float32),
        pltpu.VMEM((CHUNK_PAIRS, DIM), jnp.float32),
        pltpu.SemaphoreType.DMA,
    ],
)(_sc_body)


BBLK = 128


def _hops_body(u0_ref, m0_ref, m1_ref, m2_ref, m3_ref, twt_ref, tb_ref, u_ref):
    u = u0_ref[...]
    twt = twt_ref[...]
    tb = tb_ref[...]
    ms = (m0_ref, m1_ref, m2_ref, m3_ref)
    for h in range(HOPS):
        mh = ms[h][...]
        scores = jnp.sum(mh * u[:, None, :], axis=2)
        smax = jnp.max(scores, axis=1, keepdims=True)
        e = jnp.exp(scores - smax)
        p = e / jnp.sum(e, axis=1, keepdims=True)
        mc = ms[h + 1][...]
        o = jnp.sum(mc * p[:, :, None], axis=1)
        z = jnp.dot(u, twt, preferred_element_type=jnp.float32) + tb
        t = 1.0 / (1.0 + jnp.exp(-z))
        u = (1.0 - t) * u + o * t
    u_ref[...] = u


VBLK = 2048
NV = (VOCAB + VBLK - 1) // VBLK


def _mm_body_a(u_ref, c3_ref, o_ref):
    o_ref[...] = lax.dot_general(
        c3_ref[...], u_ref[...], (((1,), (1,)), ((), ())),
        preferred_element_type=jnp.float32)


def _mm_body_b(u_ref, c3_ref, _ahat_ref, o_ref):
    o_ref[...] = lax.dot_general(
        c3_ref[...], u_ref[...], (((1,), (1,)), ((), ())),
        preferred_element_type=jnp.float32)


def _hops(u0, m0, m1, m2, m3, twt, tb2):
    mspec = pl.BlockSpec((BBLK, MEM, DIM), lambda b: (b, 0, 0))
    return pl.pallas_call(
        _hops_body,
        grid=(BH // BBLK,),
        in_specs=[
            pl.BlockSpec((BBLK, DIM), lambda b: (b, 0)),
            mspec, mspec, mspec, mspec,
            pl.BlockSpec((DIM, DIM), lambda b: (0, 0)),
            pl.BlockSpec((1, DIM), lambda b: (0, 0)),
        ],
        out_specs=pl.BlockSpec((BBLK, DIM), lambda b: (b, 0)),
        out_shape=jax.ShapeDtypeStruct((BH, DIM), jnp.float32),
    )(u0, m0.reshape(BH, MEM, DIM), m1.reshape(BH, MEM, DIM),
      m2.reshape(BH, MEM, DIM), m3.reshape(BH, MEM, DIM), twt, tb2)


def kernel(story, query, C0, C1, C2, C3, Tk_w, Tk_b):
    enc = jnp.asarray(_ENC)
    twt = Tk_w.T
    tb2 = Tk_b.reshape(1, DIM)

    scs, us = [], []
    for i in range(NSPLIT):
        story_i = story[i * BH:(i + 1) * BH].reshape(-1).astype(jnp.int32)
        query_i = query[i * BH:(i + 1) * BH].reshape(-1).astype(jnp.int32)
        scs.append(_sc_gather(story_i, query_i, C0, C1, C2, C3, enc))
        us.append(_hops(scs[i][4], scs[i][0], scs[i][1], scs[i][2],
                        scs[i][3], twt, tb2))

    a_hat_t = pl.pallas_call(
        _mm_body_a,
        grid=(NV,),
        in_specs=[
            pl.BlockSpec((BH, DIM), lambda j: (0, 0)),
            pl.BlockSpec((VBLK, DIM), lambda j: (j, 0)),
        ],
        out_specs=pl.BlockSpec((VBLK, BH), lambda j: (j, 0)),
        out_shape=jax.ShapeDtypeStruct((VOCAB, B), jnp.float32),
    )(us[0], C3)
    for i in range(1, NSPLIT):
        a_hat_t = pl.pallas_call(
            _mm_body_b,
            grid=(NV,),
            in_specs=[
                pl.BlockSpec((BH, DIM), lambda j: (0, 0)),
                pl.BlockSpec((VBLK, DIM), lambda j: (j, 0)),
                pl.BlockSpec(memory_space=pl.ANY),
            ],
            out_specs=pl.BlockSpec((VBLK, BH), lambda j, i=i: (j, i)),
            out_shape=jax.ShapeDtypeStruct((VOCAB, B), jnp.float32),
            input_output_aliases={2: 0},
        )(us[i], C3, a_hat_t)
    return a_hat_t.T
